# Initial kernel scaffold; baseline (speedup 1.0000x reference)
import jax
import jax.numpy as jnp
from jax.experimental import pallas as pl


def kernel(pred, target):
    raise NotImplementedError("write your pallas kernel here")


# speedup vs baseline: 1.1982x; 1.1982x over previous
import functools

import jax
import jax.numpy as jnp
from jax import lax
from jax.experimental import pallas as pl
from jax.experimental.pallas import tpu as pltpu
from jax.experimental.pallas import tpu_sc as plsc

_TOP_K = 5
_BATCH = 1024
_VOCAB = 100000

_NC = 2
_NS = 16
_L = 16
_NW = _NC * _NS
_BPW = _BATCH // _NW

_BV = 2048
_NVB = (_VOCAB + _BV - 1) // _BV


def _gather_body(pred_flat_hbm, target_hbm, t_hbm, tgt_v, idx_v, out_v, sem):
    wid = lax.axis_index("s") * _NC + lax.axis_index("c")
    base = wid * _BPW
    pltpu.sync_copy(target_hbm.at[pl.ds(base, _BPW)], tgt_v)
    for k in range(_BPW // _L):
        row = base + k * _L + lax.iota(jnp.int32, _L)
        idx_v[pl.ds(k * _L, _L)] = tgt_v[pl.ds(k * _L, _L)] + row * _VOCAB
    pltpu.async_copy(pred_flat_hbm.at[idx_v], out_v, sem).wait()
    pltpu.sync_copy(out_v, t_hbm.at[pl.ds(base, _BPW)])


@functools.cache
def _make_gather():
    return pl.kernel(
        _gather_body,
        out_type=jax.ShapeDtypeStruct((_BATCH,), jnp.float32),
        mesh=plsc.VectorSubcoreMesh(
            core_axis_name="c", subcore_axis_name="s",
            num_cores=_NC, num_subcores=_NS,
        ),
        scratch_types=[
            pltpu.VMEM((_BPW,), jnp.int32),
            pltpu.VMEM((_BPW,), jnp.int32),
            pltpu.VMEM((_BPW,), jnp.float32),
            pltpu.SemaphoreType.DMA,
        ],
    )


def _count_body(t_ref, tgt_ref, pred_ref, out_ref, acc_ref):
    vb = pl.program_id(0)
    x = pred_ref[...]
    t = t_ref[...]
    tgt = tgt_ref[...]
    cols = vb * _BV + lax.broadcasted_iota(jnp.int32, (_BATCH, _BV), 1)
    thr = jnp.where(x > t, _VOCAB, jnp.where(x == t, tgt, 0))
    cnt = jnp.sum((cols < thr).astype(jnp.int32), axis=1, keepdims=True)

    @pl.when(vb == 0)
    def _init():
        acc_ref[...] = cnt

    @pl.when(vb != 0)
    def _accum():
        acc_ref[...] = acc_ref[...] + cnt

    @pl.when(vb == _NVB - 1)
    def _finish():
        correct = (acc_ref[...] < _TOP_K).astype(jnp.float32)
        out_ref[0, 0] = 100.0 * jnp.sum(correct) / jnp.float32(_BATCH)


_count = pl.pallas_call(
    _count_body,
    grid=(_NVB,),
    in_specs=[
        pl.BlockSpec((_BATCH, 1), lambda vb: (0, 0)),
        pl.BlockSpec((_BATCH, 1), lambda vb: (0, 0)),
        pl.BlockSpec((_BATCH, _BV), lambda vb: (0, vb)),
    ],
    out_specs=pl.BlockSpec(memory_space=pltpu.SMEM),
    out_shape=jax.ShapeDtypeStruct((1, 1), jnp.float32),
    scratch_shapes=[pltpu.VMEM((_BATCH, 1), jnp.int32)],
)


@jax.jit
def kernel(pred, target):
    target = target.astype(jnp.int32)
    t = _make_gather()(pred.reshape(-1), target)
    acc = _count(t.reshape(_BATCH, 1), target.reshape(_BATCH, 1), pred)
    return acc[0, 0]


# speedup vs baseline: 1.2113x; 1.0109x over previous
import functools

import jax
import jax.numpy as jnp
from jax import lax
from jax.experimental import pallas as pl
from jax.experimental.pallas import tpu as pltpu
from jax.experimental.pallas import tpu_sc as plsc

_TOP_K = 5
_BATCH = 1024
_VOCAB = 100000

_NC = 2
_NS = 16
_L = 16
_NW = _NC * _NS
_BPW = _BATCH // _NW

_BR = 32
_NRB = _BATCH // _BR


def _gather_body(pred_flat_hbm, target_hbm, t_hbm, tgt_v, idx_v, out_v, sem):
    wid = lax.axis_index("s") * _NC + lax.axis_index("c")
    base = wid * _BPW
    pltpu.sync_copy(target_hbm.at[pl.ds(base, _BPW)], tgt_v)
    for k in range(_BPW // _L):
        row = base + k * _L + lax.iota(jnp.int32, _L)
        idx_v[pl.ds(k * _L, _L)] = tgt_v[pl.ds(k * _L, _L)] + row * _VOCAB
    pltpu.async_copy(pred_flat_hbm.at[idx_v], out_v, sem).wait()
    pltpu.sync_copy(out_v, t_hbm.at[pl.ds(base, _BPW)])


@functools.cache
def _make_gather():
    return pl.kernel(
        _gather_body,
        out_type=jax.ShapeDtypeStruct((_BATCH,), jnp.float32),
        mesh=plsc.VectorSubcoreMesh(
            core_axis_name="c", subcore_axis_name="s",
            num_cores=_NC, num_subcores=_NS,
        ),
        scratch_types=[
            pltpu.VMEM((_BPW,), jnp.int32),
            pltpu.VMEM((_BPW,), jnp.int32),
            pltpu.VMEM((_BPW,), jnp.float32),
            pltpu.SemaphoreType.DMA,
        ],
    )


def _count_body(t_ref, tgt_ref, pred_ref, out_ref):
    rb = pl.program_id(0)
    x = pred_ref[...]
    t = t_ref[...]
    tgt = tgt_ref[...]
    cols = lax.broadcasted_iota(jnp.int32, (_BR, _VOCAB), 1)
    thr = jnp.where(x > t, _VOCAB, jnp.where(x == t, tgt, 0))
    cnt = jnp.sum((cols < thr).astype(jnp.int32), axis=1, keepdims=True)
    correct = jnp.sum((cnt < _TOP_K).astype(jnp.float32))

    @pl.when(rb == 0)
    def _init():
        out_ref[0, 0] = 0.0

    out_ref[0, 0] += 100.0 * correct / jnp.float32(_BATCH)


_count = pl.pallas_call(
    _count_body,
    grid=(_NRB,),
    in_specs=[
        pl.BlockSpec((_BR, 1), lambda rb: (rb, 0)),
        pl.BlockSpec((_BR, 1), lambda rb: (rb, 0)),
        pl.BlockSpec((_BR, _VOCAB), lambda rb: (rb, 0)),
    ],
    out_specs=pl.BlockSpec(memory_space=pltpu.SMEM),
    out_shape=jax.ShapeDtypeStruct((1, 1), jnp.float32),
    compiler_params=pltpu.CompilerParams(
        dimension_semantics=(pltpu.ARBITRARY,),
        vmem_limit_bytes=100 * 1024 * 1024,
    ),
)


@jax.jit
def kernel(pred, target):
    target = target.astype(jnp.int32)
    t = _make_gather()(pred.reshape(-1), target)
    acc = _count(t.reshape(_BATCH, 1), target.reshape(_BATCH, 1), pred)
    return acc[0, 0]
